# baseline (speedup 1.0000x reference)
import functools

import jax
import jax.numpy as jnp
from jax import lax
from jax.experimental import pallas as pl
from jax.experimental.pallas import tpu as pltpu
from jax.experimental.pallas import tpu_sc as plsc

N = 10000
F_IN = 128
WIDTH = 144
NUM_GRAPHS = 64
NUM_CLASSES = 112
TCOLS = 160
NROWS = 10240
E_RAW = 320000
E_TOT = E_RAW + N
SC_CORES = 2
SC_SUBCORES = 16
CHUNK = 48
EPW = 10368
NCH = EPW // CHUNK
EPAD = SC_CORES * SC_SUBCORES * EPW
ROWS_PT = NROWS // SC_SUBCORES
BLK = 400
GRID = N // BLK
EPS = 1e-5


def _edge_body(t_hbm, ad_hbm, src_hbm, dst_hbm, imap_hbm, acc_hbm,
               shared, tb0, tb1, tb2, ab0, ab1, ab2,
               is0a, is0b, is1a, is1b, is2a, is2b,
               id0a, id0b, id1a, id1b, id2a, id2b,
               iprime, imapv,
               st0, st1, st2, sa0, sa1, sa2, ss0, ss1, ss2, si0, si1, si2):
    cid = lax.axis_index("c")
    sid = lax.axis_index("s")
    row0 = sid * ROWS_PT
    tb = (tb0, tb1, tb2)
    ab = (ab0, ab1, ab2)
    isl = ((is0a, is0b), (is1a, is1b), (is2a, is2b))
    idl = ((id0a, id0b), (id1a, id1b), (id2a, id2b))
    st = (st0, st1, st2)
    sa = (sa0, sa1, sa2)
    ss = (ss0, ss1, ss2)
    si = (si0, si1, si2)

    def _zrow(r, carry):
        for j in range(TCOLS // 16):
            tb0[r, pl.ds(j * 16, 16)] = jnp.zeros((16,), jnp.float32)
            tb1[r, pl.ds(j * 16, 16)] = jnp.zeros((16,), jnp.float32)
            tb2[r, pl.ds(j * 16, 16)] = jnp.zeros((16,), jnp.float32)
        return carry
    lax.fori_loop(0, CHUNK, _zrow, 0)
    for j in range(CHUNK // 16):
        iprime[pl.ds(j * 16, 16)] = jnp.full((16,), N, jnp.int32)

    for k in range(ROWS_PT // CHUNK):
        pltpu.sync_copy(tb0, shared.at[pl.ds(row0 + k * CHUNK, CHUNK)])
    rem = ROWS_PT % CHUNK
    if rem:
        pltpu.sync_copy(tb0.at[pl.ds(0, rem)],
                        shared.at[pl.ds(row0 + (ROWS_PT // CHUNK) * CHUNK, rem)])
    plsc.subcore_barrier()

    pltpu.sync_copy(imap_hbm, imapv)
    imaps = [imapv[pl.ds(j * 16, 16)] for j in range(WIDTH // 16)]

    ebase = (cid * SC_SUBCORES + sid) * EPW

    def _compute(x):
        def _edge(e, c2):
            a = tb[x][e, pl.ds(WIDTH, 16)] + ab[x][e, :]
            s = jnp.exp(jnp.maximum(a, 0.0) + 0.2 * jnp.minimum(a, 0.0))
            tb[x][e, pl.ds(WIDTH, 16)] = s
            for j in range(WIDTH // 16):
                m = jnp.take_along_axis(s, imaps[j], axis=0,
                                        mode="promise_in_bounds")
                tb[x][e, pl.ds(j * 16, 16)] = m * tb[x][e, pl.ds(j * 16, 16)]
            return c2
        lax.fori_loop(0, CHUNK, _edge, 0, unroll=2)

    def _idx_copy(c, x, p, sem):
        b = ebase + c * CHUNK
        pltpu.async_copy(src_hbm.at[pl.ds(b, CHUNK)], isl[x][p], sem)
        pltpu.async_copy(dst_hbm.at[pl.ds(b, CHUNK)], idl[x][p], sem)

    def _idx_wait(x, p, sem):
        pltpu.make_async_copy(src_hbm.at[pl.ds(0, CHUNK)], isl[x][p], sem).wait()
        pltpu.make_async_copy(dst_hbm.at[pl.ds(0, CHUNK)], idl[x][p], sem).wait()

    def _gather(x, p):
        pltpu.async_copy(t_hbm.at[isl[x][p]], tb[x], st[x])
        pltpu.async_copy(ad_hbm.at[idl[x][p]], ab[x], sa[x])

    def _gather_wait(x):
        pltpu.make_async_copy(t_hbm.at[isl[x][0]], tb[x], st[x]).wait()
        pltpu.make_async_copy(ad_hbm.at[idl[x][0]], ab[x], sa[x]).wait()

    def _scatter(x, p, sem):
        pltpu.async_copy(tb[x], shared.at[idl[x][p]], sem, add=True)

    def _scatter_wait(x):
        pltpu.make_async_copy(tb[x], shared.at[iprime], ss[x]).wait()

    pltpu.sync_copy(src_hbm.at[pl.ds(ebase, CHUNK)], is0a)
    pltpu.sync_copy(dst_hbm.at[pl.ds(ebase, CHUNK)], id0a)
    pltpu.sync_copy(src_hbm.at[pl.ds(ebase + CHUNK, CHUNK)], is1a)
    pltpu.sync_copy(dst_hbm.at[pl.ds(ebase + CHUNK, CHUNK)], id1a)
    _idx_copy(2, 2, 0, si[2])
    _gather(0, 0)
    _gather(1, 0)

    def _visit(g, k, first=False):
        x = k % 3
        p = (k // 3) % 2
        prev = (x + 2) % 3
        p2 = ((k + 2) // 3) % 2
        _gather_wait(x)
        _idx_copy((g + 3) % NCH, x, 1 - p, si[x])
        _compute(x)
        _scatter(x, p, ss[x])
        if not first:
            _scatter_wait(prev)
        _idx_wait(prev, p2, si[prev])
        _gather(prev, p2)

    for k in range(6):
        _visit(k, k, first=(k == 0))

    def _six(i, carry):
        g0 = 6 + i * 6
        for k in range(6):
            _visit(g0 + k, k)
        return carry
    lax.fori_loop(0, NCH // 6 - 1, _six, 0)

    _gather_wait(0)
    _gather_wait(1)
    _scatter_wait(2)
    _idx_wait(2, 0, si[2])

    plsc.subcore_barrier()
    pltpu.sync_copy(shared.at[pl.ds(row0, ROWS_PT)],
                    acc_hbm.at[cid, pl.ds(row0, ROWS_PT), :])


@functools.lru_cache(maxsize=1)
def _build_edge_kernel():
    return functools.partial(
        pl.kernel,
        out_type=jax.ShapeDtypeStruct((SC_CORES, NROWS, TCOLS), jnp.float32),
        mesh=plsc.VectorSubcoreMesh(core_axis_name="c", subcore_axis_name="s",
                                    num_cores=SC_CORES,
                                    num_subcores=SC_SUBCORES),
        scratch_types=(
            [pltpu.VMEM_SHARED((NROWS, TCOLS), jnp.float32)]
            + [pltpu.VMEM((CHUNK, TCOLS), jnp.float32)] * 3
            + [pltpu.VMEM((CHUNK, 16), jnp.float32)] * 3
            + [pltpu.VMEM((CHUNK,), jnp.int32)] * 12
            + [pltpu.VMEM((CHUNK,), jnp.int32)]
            + [pltpu.VMEM((TCOLS,), jnp.int32)]
            + [pltpu.SemaphoreType.DMA] * 12
        ),
        compiler_params=pltpu.CompilerParams(use_tc_tiling_on_sc=False),
    )(_edge_body)


def _edge_call(t, ad16, src, dst, imap):
    return _build_edge_kernel()(t, ad16, src, dst, imap)


def _emit_tables(h, asf, adf, t_ref, ad_ref, heads, ch):
    col_head = lax.broadcasted_iota(jnp.int32, (WIDTH, heads), 0) // ch
    hid = lax.broadcasted_iota(jnp.int32, (WIDTH, heads), 1)
    m = (col_head == hid).astype(jnp.float32)
    as_ = jnp.dot(h * asf, m, preferred_element_type=jnp.float32,
                 precision=lax.Precision.HIGHEST)
    ad_ = jnp.dot(h * adf, m, preferred_element_type=jnp.float32,
                 precision=lax.Precision.HIGHEST)
    zp = jnp.zeros((h.shape[0], 16 - heads), jnp.float32)
    t_ref[...] = jnp.concatenate([h, as_, zp], axis=1)
    ad_ref[...] = jnp.concatenate([ad_, zp], axis=1)


def _pre0_body(x_ref, wemb_ref, bemb_ref, w_ref, asf_ref, adf_ref,
               t_ref, ad_ref, *, heads, ch):
    x0 = jnp.dot(x_ref[...], wemb_ref[...],
                 preferred_element_type=jnp.float32) + bemb_ref[...]
    h = jnp.dot(x0, w_ref[...], preferred_element_type=jnp.float32)
    _emit_tables(h, asf_ref[...], adf_ref[...], t_ref, ad_ref, heads, ch)


def _prei_body(z_ref, ps_ref, g_ref, b_ref, w_ref, asf_ref, adf_ref,
               t_ref, ad_ref, *, heads, ch):
    sums = jnp.sum(ps_ref[...][:, 0, :], axis=0, keepdims=True)
    mu = sums[:, :WIDTH] / N
    var = sums[:, WIDTH:] / N - mu * mu
    scale = g_ref[...] * lax.rsqrt(var + EPS)
    shift = b_ref[...] - mu * scale
    xn = z_ref[...] * scale + shift
    h = jnp.dot(xn, w_ref[...], preferred_element_type=jnp.float32)
    _emit_tables(h, asf_ref[...], adf_ref[...], t_ref, ad_ref, heads, ch)


def _post_body(acc_ref, bias_ref, z_ref, ps_ref, *, heads, ch):
    acc = acc_ref[0] + acc_ref[1]
    num = acc[:, :WIDTH]
    den = acc[:, WIDTH:WIDTH + heads]
    hid = lax.broadcasted_iota(jnp.int32, (heads, WIDTH), 0)
    col_head = lax.broadcasted_iota(jnp.int32, (heads, WIDTH), 1) // ch
    bexp = (hid == col_head).astype(jnp.float32)
    dex = jnp.dot(den, bexp, preferred_element_type=jnp.float32,
                 precision=lax.Precision.HIGHEST)
    z = jnp.maximum(num / (dex + 1e-16) + bias_ref[...], 0.0)
    z_ref[...] = z
    ps_ref[...] = jnp.concatenate(
        [jnp.sum(z, axis=0, keepdims=True),
         jnp.sum(z * z, axis=0, keepdims=True)], axis=1)[None]


def _bn_small(x, g, b):
    mu = jnp.mean(x, axis=0, keepdims=True)
    var = jnp.mean(x * x, axis=0, keepdims=True) - mu * mu
    return g * (x - mu) * lax.rsqrt(var + EPS) + b


def _head_body(z_ref, ps_ref, g_ref, b_ref, batch_ref,
               w0_ref, b0_ref, g0_ref, t0_ref,
               w1_ref, b1_ref, g1_ref, t1_ref,
               w2_ref, b2_ref, out_ref, pooled, cnt):
    i = pl.program_id(0)

    @pl.when(i == 0)
    def _init():
        pooled[...] = jnp.zeros_like(pooled)
        cnt[...] = jnp.zeros_like(cnt)

    sums = jnp.sum(ps_ref[...][:, 0, :], axis=0, keepdims=True)
    mu = sums[:, :WIDTH] / N
    var = sums[:, WIDTH:] / N - mu * mu
    scale = g_ref[...] * lax.rsqrt(var + EPS)
    shift = b_ref[...] - mu * scale
    zn = z_ref[...] * scale + shift

    bvec = batch_ref[0]
    gid = lax.broadcasted_iota(jnp.int32, (NUM_GRAPHS, BLK), 0)
    onehot_t = (gid == bvec).astype(jnp.float32)
    pooled[...] += jnp.dot(onehot_t, zn, preferred_element_type=jnp.float32,
                 precision=lax.Precision.HIGHEST)
    cnt[...] += jnp.dot(onehot_t, jnp.ones((BLK, 1), jnp.float32),
                        preferred_element_type=jnp.float32,
                 precision=lax.Precision.HIGHEST)

    @pl.when(i == GRID - 1)
    def _finish():
        pool = pooled[...] / jnp.maximum(cnt[...], 1.0)
        h0 = jnp.dot(pool, w0_ref[...],
                     preferred_element_type=jnp.float32) + b0_ref[...]
        h0 = jnp.maximum(_bn_small(h0, g0_ref[...], t0_ref[...]), 0.0)
        h1 = jnp.dot(h0, w1_ref[...],
                     preferred_element_type=jnp.float32) + b1_ref[...]
        h1 = jnp.maximum(_bn_small(h1, g1_ref[...], t1_ref[...]), 0.0)
        out_ref[...] = jnp.dot(h1, w2_ref[...],
                     preferred_element_type=jnp.float32) + b2_ref[...]


def _full(shape):
    return pl.BlockSpec(shape, lambda i: tuple(0 for _ in shape))


def _rows(cols):
    return pl.BlockSpec((BLK, cols), lambda i: (i, 0))


def _pre0_call(x, wemb, bemb, w, asf, adf, heads, ch):
    return pl.pallas_call(
        functools.partial(_pre0_body, heads=heads, ch=ch),
        grid=(GRID,),
        in_specs=[_rows(F_IN), _full((F_IN, WIDTH)), _full((1, WIDTH)),
                  _full((WIDTH, WIDTH)), _full((1, WIDTH)), _full((1, WIDTH))],
        out_specs=[_rows(TCOLS), _rows(16)],
        out_shape=[jax.ShapeDtypeStruct((N, TCOLS), jnp.float32),
                   jax.ShapeDtypeStruct((N, 16), jnp.float32)],
    )(x, wemb, bemb, w, asf, adf)


def _prei_call(z, ps, g, b, w, asf, adf, heads, ch):
    return pl.pallas_call(
        functools.partial(_prei_body, heads=heads, ch=ch),
        grid=(GRID,),
        in_specs=[_rows(WIDTH), _full((GRID, 1, 2 * WIDTH)), _full((1, WIDTH)),
                  _full((1, WIDTH)), _full((WIDTH, WIDTH)),
                  _full((1, WIDTH)), _full((1, WIDTH))],
        out_specs=[_rows(TCOLS), _rows(16)],
        out_shape=[jax.ShapeDtypeStruct((N, TCOLS), jnp.float32),
                   jax.ShapeDtypeStruct((N, 16), jnp.float32)],
    )(z, ps, g, b, w, asf, adf)


def _post_call(acc, bias, heads, ch):
    return pl.pallas_call(
        functools.partial(_post_body, heads=heads, ch=ch),
        grid=(GRID,),
        in_specs=[pl.BlockSpec((SC_CORES, BLK, TCOLS), lambda i: (0, i, 0)),
                  _full((1, WIDTH))],
        out_specs=[_rows(WIDTH),
                   pl.BlockSpec((1, 1, 2 * WIDTH), lambda i: (i, 0, 0))],
        out_shape=[jax.ShapeDtypeStruct((N, WIDTH), jnp.float32),
                   jax.ShapeDtypeStruct((GRID, 1, 2 * WIDTH), jnp.float32)],
    )(acc, bias)


def _head_call(z, ps, g, b, batch3, mp):
    return pl.pallas_call(
        _head_body,
        grid=(GRID,),
        in_specs=[_rows(WIDTH), _full((GRID, 1, 2 * WIDTH)), _full((1, WIDTH)),
                  _full((1, WIDTH)),
                  pl.BlockSpec((1, 1, BLK), lambda i: (i, 0, 0)),
                  _full((WIDTH, 72)), _full((1, 72)), _full((1, 72)),
                  _full((1, 72)),
                  _full((72, 36)), _full((1, 36)), _full((1, 36)),
                  _full((1, 36)),
                  _full((36, NUM_CLASSES)), _full((1, NUM_CLASSES))],
        out_specs=pl.BlockSpec((NUM_GRAPHS, NUM_CLASSES), lambda i: (0, 0)),
        out_shape=jax.ShapeDtypeStruct((NUM_GRAPHS, NUM_CLASSES), jnp.float32),
        scratch_shapes=[pltpu.VMEM((NUM_GRAPHS, WIDTH), jnp.float32),
                        pltpu.VMEM((NUM_GRAPHS, 1), jnp.float32)],
    )(z, ps, g, b, batch3, *mp)


def kernel(x, edge_index, batch, params):
    p = params
    loops = jnp.arange(N, dtype=jnp.int32)
    npad = EPAD - E_TOT
    src = jnp.concatenate([edge_index[0].astype(jnp.int32), loops,
                           jnp.zeros((npad,), jnp.int32)])
    dst = jnp.concatenate([edge_index[1].astype(jnp.int32), loops,
                           jnp.full((npad,), N, jnp.int32)])
    imap_multi = jnp.arange(TCOLS, dtype=jnp.int32) // 18
    imap_single = jnp.zeros((TCOLS,), jnp.int32)
    batch3 = batch.astype(jnp.int32).reshape(GRID, 1, BLK)

    cfg = [(8, 18), (8, 18), (8, 18), (1, WIDTH)]
    z = None
    ps = None
    for i, (heads, ch) in enumerate(cfg):
        asf = p[f'gat{i}_as'].reshape(1, WIDTH)
        adf = p[f'gat{i}_ad'].reshape(1, WIDTH)
        w = p[f'gat{i}_W']
        if i == 0:
            t, ad16 = _pre0_call(x, p['W_emb'], p['b_emb'].reshape(1, WIDTH),
                                 w, asf, adf, heads, ch)
        else:
            t, ad16 = _prei_call(z, ps, p[f'bn{i-1}_g'].reshape(1, WIDTH),
                                 p[f'bn{i-1}_b'].reshape(1, WIDTH),
                                 w, asf, adf, heads, ch)
        imap = imap_multi if heads > 1 else imap_single
        acc = _edge_call(t, ad16, src, dst, imap)
        z, ps = _post_call(acc, p[f'gat{i}_b'].reshape(1, WIDTH), heads, ch)

    mp = [p['mlp_W0'], p['mlp_b0'].reshape(1, 72),
          p['mlp_g0'].reshape(1, 72), p['mlp_beta0'].reshape(1, 72),
          p['mlp_W1'], p['mlp_b1'].reshape(1, 36),
          p['mlp_g1'].reshape(1, 36), p['mlp_beta1'].reshape(1, 36),
          p['mlp_W2'], p['mlp_b2'].reshape(1, NUM_CLASSES)]
    return _head_call(z, ps, p['bn3_g'].reshape(1, WIDTH),
                      p['bn3_b'].reshape(1, WIDTH), batch3, mp)


# speedup vs baseline: 1.0038x; 1.0038x over previous
import functools

import jax
import jax.numpy as jnp
from jax import lax
from jax.experimental import pallas as pl
from jax.experimental.pallas import tpu as pltpu
from jax.experimental.pallas import tpu_sc as plsc

N = 10000
F_IN = 128
WIDTH = 144
NUM_GRAPHS = 64
NUM_CLASSES = 112
TCOLS = 160
NROWS = 10240
E_RAW = 320000
E_TOT = E_RAW + N
SC_CORES = 2
SC_SUBCORES = 16
CHUNK = 48
EPW = 10368
NCH = EPW // CHUNK
EPAD = SC_CORES * SC_SUBCORES * EPW
ROWS_PT = NROWS // SC_SUBCORES
BLK = 400
GRID = N // BLK
EPS = 1e-5


def _edge_body(t_hbm, ad_hbm, src_hbm, dst_hbm, imap_hbm, acc_hbm,
               shared, tb0, tb1, tb2, ab0, ab1, ab2,
               is0a, is0b, is1a, is1b, is2a, is2b,
               id0a, id0b, id1a, id1b, id2a, id2b,
               iprime, imapv,
               st0, st1, st2, sa0, sa1, sa2, ss0, ss1, ss2, si0, si1, si2):
    cid = lax.axis_index("c")
    sid = lax.axis_index("s")
    row0 = sid * ROWS_PT
    tb = (tb0, tb1, tb2)
    ab = (ab0, ab1, ab2)
    isl = ((is0a, is0b), (is1a, is1b), (is2a, is2b))
    idl = ((id0a, id0b), (id1a, id1b), (id2a, id2b))
    st = (st0, st1, st2)
    sa = (sa0, sa1, sa2)
    ss = (ss0, ss1, ss2)
    si = (si0, si1, si2)

    def _zrow(r, carry):
        for j in range(TCOLS // 16):
            tb0[r, pl.ds(j * 16, 16)] = jnp.zeros((16,), jnp.float32)
            tb1[r, pl.ds(j * 16, 16)] = jnp.zeros((16,), jnp.float32)
            tb2[r, pl.ds(j * 16, 16)] = jnp.zeros((16,), jnp.float32)
        return carry
    lax.fori_loop(0, CHUNK, _zrow, 0)
    for j in range(CHUNK // 16):
        iprime[pl.ds(j * 16, 16)] = jnp.full((16,), N, jnp.int32)

    for k in range(ROWS_PT // CHUNK):
        pltpu.sync_copy(tb0, shared.at[pl.ds(row0 + k * CHUNK, CHUNK)])
    rem = ROWS_PT % CHUNK
    if rem:
        pltpu.sync_copy(tb0.at[pl.ds(0, rem)],
                        shared.at[pl.ds(row0 + (ROWS_PT // CHUNK) * CHUNK, rem)])
    plsc.subcore_barrier()

    pltpu.sync_copy(imap_hbm, imapv)
    imaps = [imapv[pl.ds(j * 16, 16)] for j in range(WIDTH // 16)]

    ebase = (cid * SC_SUBCORES + sid) * EPW

    def _compute(x):
        def _edge(e, c2):
            a = tb[x][e, pl.ds(WIDTH, 16)] + ab[x][e, :]
            s = jnp.exp(jnp.maximum(a, 0.0) + 0.2 * jnp.minimum(a, 0.0))
            tb[x][e, pl.ds(WIDTH, 16)] = s
            for j in range(WIDTH // 16):
                m = jnp.take_along_axis(s, imaps[j], axis=0,
                                        mode="promise_in_bounds")
                tb[x][e, pl.ds(j * 16, 16)] = m * tb[x][e, pl.ds(j * 16, 16)]
            return c2
        lax.fori_loop(0, CHUNK, _edge, 0)

    def _idx_copy(c, x, p, sem):
        b = ebase + c * CHUNK
        pltpu.async_copy(src_hbm.at[pl.ds(b, CHUNK)], isl[x][p], sem)
        pltpu.async_copy(dst_hbm.at[pl.ds(b, CHUNK)], idl[x][p], sem)

    def _idx_wait(x, p, sem):
        pltpu.make_async_copy(src_hbm.at[pl.ds(0, CHUNK)], isl[x][p], sem).wait()
        pltpu.make_async_copy(dst_hbm.at[pl.ds(0, CHUNK)], idl[x][p], sem).wait()

    def _gather(x, p):
        pltpu.async_copy(t_hbm.at[isl[x][p]], tb[x], st[x])
        pltpu.async_copy(ad_hbm.at[idl[x][p]], ab[x], sa[x])

    def _gather_wait(x):
        pltpu.make_async_copy(t_hbm.at[isl[x][0]], tb[x], st[x]).wait()
        pltpu.make_async_copy(ad_hbm.at[idl[x][0]], ab[x], sa[x]).wait()

    def _scatter(x, p, sem):
        pltpu.async_copy(tb[x], shared.at[idl[x][p]], sem, add=True)

    def _scatter_wait(x):
        pltpu.make_async_copy(tb[x], shared.at[iprime], ss[x]).wait()

    pltpu.sync_copy(src_hbm.at[pl.ds(ebase, CHUNK)], is0a)
    pltpu.sync_copy(dst_hbm.at[pl.ds(ebase, CHUNK)], id0a)
    pltpu.sync_copy(src_hbm.at[pl.ds(ebase + CHUNK, CHUNK)], is1a)
    pltpu.sync_copy(dst_hbm.at[pl.ds(ebase + CHUNK, CHUNK)], id1a)
    _idx_copy(2, 2, 0, si[2])
    _gather(0, 0)
    _gather(1, 0)

    def _visit(g, k, first=False):
        x = k % 3
        p = (k // 3) % 2
        prev = (x + 2) % 3
        p2 = ((k + 2) // 3) % 2
        _gather_wait(x)
        _idx_copy((g + 3) % NCH, x, 1 - p, si[x])
        _compute(x)
        _scatter(x, p, ss[x])
        if not first:
            _scatter_wait(prev)
        _idx_wait(prev, p2, si[prev])
        _gather(prev, p2)

    for k in range(6):
        _visit(k, k, first=(k == 0))

    def _six(i, carry):
        g0 = 6 + i * 6
        for k in range(6):
            _visit(g0 + k, k)
        return carry
    lax.fori_loop(0, NCH // 6 - 1, _six, 0)

    _gather_wait(0)
    _gather_wait(1)
    _scatter_wait(2)
    _idx_wait(2, 0, si[2])

    plsc.subcore_barrier()
    pltpu.sync_copy(shared.at[pl.ds(row0, ROWS_PT)],
                    acc_hbm.at[cid, pl.ds(row0, ROWS_PT), :])


@functools.lru_cache(maxsize=1)
def _build_edge_kernel():
    return functools.partial(
        pl.kernel,
        out_type=jax.ShapeDtypeStruct((SC_CORES, NROWS, TCOLS), jnp.float32),
        mesh=plsc.VectorSubcoreMesh(core_axis_name="c", subcore_axis_name="s",
                                    num_cores=SC_CORES,
                                    num_subcores=SC_SUBCORES),
        scratch_types=(
            [pltpu.VMEM_SHARED((NROWS, TCOLS), jnp.float32)]
            + [pltpu.VMEM((CHUNK, TCOLS), jnp.float32)] * 3
            + [pltpu.VMEM((CHUNK, 16), jnp.float32)] * 3
            + [pltpu.VMEM((CHUNK,), jnp.int32)] * 12
            + [pltpu.VMEM((CHUNK,), jnp.int32)]
            + [pltpu.VMEM((TCOLS,), jnp.int32)]
            + [pltpu.SemaphoreType.DMA] * 12
        ),
        compiler_params=pltpu.CompilerParams(use_tc_tiling_on_sc=False),
    )(_edge_body)


def _edge_call(t, ad16, src, dst, imap):
    return _build_edge_kernel()(t, ad16, src, dst, imap)


def _emit_tables(h, asf, adf, t_ref, ad_ref, heads, ch):
    col_head = lax.broadcasted_iota(jnp.int32, (WIDTH, heads), 0) // ch
    hid = lax.broadcasted_iota(jnp.int32, (WIDTH, heads), 1)
    m = (col_head == hid).astype(jnp.float32)
    as_ = jnp.dot(h * asf, m, preferred_element_type=jnp.float32,
                 precision=lax.Precision.HIGHEST)
    ad_ = jnp.dot(h * adf, m, preferred_element_type=jnp.float32,
                 precision=lax.Precision.HIGHEST)
    zp = jnp.zeros((h.shape[0], 16 - heads), jnp.float32)
    t_ref[...] = jnp.concatenate([h, as_, zp], axis=1)
    ad_ref[...] = jnp.concatenate([ad_, zp], axis=1)


def _pre0_body(x_ref, wemb_ref, bemb_ref, w_ref, asf_ref, adf_ref,
               t_ref, ad_ref, *, heads, ch):
    x0 = jnp.dot(x_ref[...], wemb_ref[...],
                 preferred_element_type=jnp.float32) + bemb_ref[...]
    h = jnp.dot(x0, w_ref[...], preferred_element_type=jnp.float32)
    _emit_tables(h, asf_ref[...], adf_ref[...], t_ref, ad_ref, heads, ch)


def _post_z(acc_ref, bias_ref, heads, ch):
    acc = acc_ref[0] + acc_ref[1]
    num = acc[:, :WIDTH]
    den = acc[:, WIDTH:WIDTH + heads]
    hid = lax.broadcasted_iota(jnp.int32, (heads, WIDTH), 0)
    col_head = lax.broadcasted_iota(jnp.int32, (heads, WIDTH), 1) // ch
    bexp = (hid == col_head).astype(jnp.float32)
    dex = jnp.dot(den, bexp, preferred_element_type=jnp.float32,
                  precision=lax.Precision.HIGHEST)
    return jnp.maximum(num / (dex + 1e-16) + bias_ref[...], 0.0)


def _bn_coeffs(stats, g_ref, b_ref):
    sums = stats[...]
    mu = sums[:, :WIDTH] / N
    var = sums[:, WIDTH:] / N - mu * mu
    scale = g_ref[...] * lax.rsqrt(var + EPS)
    shift = b_ref[...] - mu * scale
    return scale, shift


def _fuse_body(acc_ref, bias_ref, g_ref, b_ref, w_ref, asf_ref, adf_ref,
               t_ref, ad_ref, zbuf, stats, *, heads, ch, heads2, ch2):
    s_idx = pl.program_id(0)
    b_idx = pl.program_id(1)

    @pl.when(s_idx == 0)
    def _pass0():
        z = _post_z(acc_ref, bias_ref, heads, ch)
        zbuf[pl.ds(b_idx * BLK, BLK), :] = z
        pstat = jnp.concatenate(
            [jnp.sum(z, axis=0, keepdims=True),
             jnp.sum(z * z, axis=0, keepdims=True)], axis=1)

        @pl.when(b_idx == 0)
        def _first():
            stats[...] = pstat

        @pl.when(b_idx > 0)
        def _rest():
            stats[...] += pstat

    @pl.when(s_idx == 1)
    def _pass1():
        scale, shift = _bn_coeffs(stats, g_ref, b_ref)
        xn = zbuf[pl.ds(b_idx * BLK, BLK), :] * scale + shift
        h = jnp.dot(xn, w_ref[...], preferred_element_type=jnp.float32)
        _emit_tables(h, asf_ref[...], adf_ref[...], t_ref, ad_ref, heads2, ch2)


def _bn_small(x, g, b):
    mu = jnp.mean(x, axis=0, keepdims=True)
    var = jnp.mean(x * x, axis=0, keepdims=True) - mu * mu
    return g * (x - mu) * lax.rsqrt(var + EPS) + b


def _head_body(acc_ref, bias_ref, g_ref, b_ref, batch_ref,
               w0_ref, b0_ref, g0_ref, t0_ref,
               w1_ref, b1_ref, g1_ref, t1_ref,
               w2_ref, b2_ref, out_ref, zbuf, stats, pooled, cnt,
               *, heads, ch):
    s_idx = pl.program_id(0)
    b_idx = pl.program_id(1)

    @pl.when(s_idx == 0)
    def _pass0():
        z = _post_z(acc_ref, bias_ref, heads, ch)
        zbuf[pl.ds(b_idx * BLK, BLK), :] = z
        pstat = jnp.concatenate(
            [jnp.sum(z, axis=0, keepdims=True),
             jnp.sum(z * z, axis=0, keepdims=True)], axis=1)

        @pl.when(b_idx == 0)
        def _first():
            stats[...] = pstat
            pooled[...] = jnp.zeros_like(pooled)
            cnt[...] = jnp.zeros_like(cnt)

        @pl.when(b_idx > 0)
        def _rest():
            stats[...] += pstat

    @pl.when(s_idx == 1)
    def _pass1():
        scale, shift = _bn_coeffs(stats, g_ref, b_ref)
        zn = zbuf[pl.ds(b_idx * BLK, BLK), :] * scale + shift

        bvec = batch_ref[0]
        gid = lax.broadcasted_iota(jnp.int32, (NUM_GRAPHS, BLK), 0)
        onehot_t = (gid == bvec).astype(jnp.float32)
        pooled[...] += jnp.dot(onehot_t, zn,
                               preferred_element_type=jnp.float32,
                               precision=lax.Precision.HIGHEST)
        cnt[...] += jnp.dot(onehot_t, jnp.ones((BLK, 1), jnp.float32),
                            preferred_element_type=jnp.float32,
                            precision=lax.Precision.HIGHEST)

        @pl.when(b_idx == GRID - 1)
        def _finish():
            pool = pooled[...] / jnp.maximum(cnt[...], 1.0)
            h0 = jnp.dot(pool, w0_ref[...],
                         preferred_element_type=jnp.float32) + b0_ref[...]
            h0 = jnp.maximum(_bn_small(h0, g0_ref[...], t0_ref[...]), 0.0)
            h1 = jnp.dot(h0, w1_ref[...],
                         preferred_element_type=jnp.float32) + b1_ref[...]
            h1 = jnp.maximum(_bn_small(h1, g1_ref[...], t1_ref[...]), 0.0)
            out_ref[...] = jnp.dot(h1, w2_ref[...],
                                   preferred_element_type=jnp.float32) + b2_ref[...]


def _full(shape):
    return pl.BlockSpec(shape, lambda s, b: tuple(0 for _ in shape))


def _pre0_call(x, wemb, bemb, w, asf, adf, heads, ch):
    rows = lambda cols: pl.BlockSpec((BLK, cols), lambda i: (i, 0))
    full = lambda shape: pl.BlockSpec(shape, lambda i: tuple(0 for _ in shape))
    return pl.pallas_call(
        functools.partial(_pre0_body, heads=heads, ch=ch),
        grid=(GRID,),
        in_specs=[rows(F_IN), full((F_IN, WIDTH)), full((1, WIDTH)),
                  full((WIDTH, WIDTH)), full((1, WIDTH)), full((1, WIDTH))],
        out_specs=[rows(TCOLS), rows(16)],
        out_shape=[jax.ShapeDtypeStruct((N, TCOLS), jnp.float32),
                   jax.ShapeDtypeStruct((N, 16), jnp.float32)],
    )(x, wemb, bemb, w, asf, adf)


def _fuse_call(acc, bias, g, b, w, asf, adf, heads, ch, heads2, ch2):
    return pl.pallas_call(
        functools.partial(_fuse_body, heads=heads, ch=ch,
                          heads2=heads2, ch2=ch2),
        grid=(2, GRID),
        in_specs=[pl.BlockSpec((SC_CORES, BLK, TCOLS),
                               lambda s, b: (0, b * (1 - s), 0)),
                  _full((1, WIDTH)), _full((1, WIDTH)), _full((1, WIDTH)),
                  _full((WIDTH, WIDTH)), _full((1, WIDTH)), _full((1, WIDTH))],
        out_specs=[pl.BlockSpec((BLK, TCOLS), lambda s, b: (b * s, 0)),
                   pl.BlockSpec((BLK, 16), lambda s, b: (b * s, 0))],
        out_shape=[jax.ShapeDtypeStruct((N, TCOLS), jnp.float32),
                   jax.ShapeDtypeStruct((N, 16), jnp.float32)],
        scratch_shapes=[pltpu.VMEM((N, WIDTH), jnp.float32),
                        pltpu.VMEM((1, 2 * WIDTH), jnp.float32)],
    )(acc, bias, g, b, w, asf, adf)


def _head_call(acc, bias, g, b, batch3, mp, heads, ch):
    return pl.pallas_call(
        functools.partial(_head_body, heads=heads, ch=ch),
        grid=(2, GRID),
        in_specs=[pl.BlockSpec((SC_CORES, BLK, TCOLS),
                               lambda s, b: (0, b * (1 - s), 0)),
                  _full((1, WIDTH)), _full((1, WIDTH)), _full((1, WIDTH)),
                  pl.BlockSpec((1, 1, BLK), lambda s, b: (b * s, 0, 0)),
                  _full((WIDTH, 72)), _full((1, 72)), _full((1, 72)),
                  _full((1, 72)),
                  _full((72, 36)), _full((1, 36)), _full((1, 36)),
                  _full((1, 36)),
                  _full((36, NUM_CLASSES)), _full((1, NUM_CLASSES))],
        out_specs=pl.BlockSpec((NUM_GRAPHS, NUM_CLASSES), lambda s, b: (0, 0)),
        out_shape=jax.ShapeDtypeStruct((NUM_GRAPHS, NUM_CLASSES), jnp.float32),
        scratch_shapes=[pltpu.VMEM((N, WIDTH), jnp.float32),
                        pltpu.VMEM((1, 2 * WIDTH), jnp.float32),
                        pltpu.VMEM((NUM_GRAPHS, WIDTH), jnp.float32),
                        pltpu.VMEM((NUM_GRAPHS, 1), jnp.float32)],
    )(acc, bias, g, b, batch3, *mp)


def kernel(x, edge_index, batch, params):
    p = params
    loops = jnp.arange(N, dtype=jnp.int32)
    npad = EPAD - E_TOT
    src = jnp.concatenate([edge_index[0].astype(jnp.int32), loops,
                           jnp.zeros((npad,), jnp.int32)])
    dst = jnp.concatenate([edge_index[1].astype(jnp.int32), loops,
                           jnp.full((npad,), N, jnp.int32)])
    imap_multi = jnp.arange(TCOLS, dtype=jnp.int32) // 18
    imap_single = jnp.zeros((TCOLS,), jnp.int32)
    batch3 = batch.astype(jnp.int32).reshape(GRID, 1, BLK)

    cfg = [(8, 18), (8, 18), (8, 18), (1, WIDTH)]
    t, ad16 = _pre0_call(x, p['W_emb'], p['b_emb'].reshape(1, WIDTH),
                         p['gat0_W'], p['gat0_as'].reshape(1, WIDTH),
                         p['gat0_ad'].reshape(1, WIDTH), *cfg[0])
    acc = _edge_call(t, ad16, src, dst, imap_multi)
    for i in range(3):
        heads, ch = cfg[i]
        heads2, ch2 = cfg[i + 1]
        t, ad16 = _fuse_call(acc, p[f'gat{i}_b'].reshape(1, WIDTH),
                             p[f'bn{i}_g'].reshape(1, WIDTH),
                             p[f'bn{i}_b'].reshape(1, WIDTH),
                             p[f'gat{i+1}_W'],
                             p[f'gat{i+1}_as'].reshape(1, WIDTH),
                             p[f'gat{i+1}_ad'].reshape(1, WIDTH),
                             heads, ch, heads2, ch2)
        acc = _edge_call(t, ad16, src, dst,
                         imap_multi if heads2 > 1 else imap_single)

    mp = [p['mlp_W0'], p['mlp_b0'].reshape(1, 72),
          p['mlp_g0'].reshape(1, 72), p['mlp_beta0'].reshape(1, 72),
          p['mlp_W1'], p['mlp_b1'].reshape(1, 36),
          p['mlp_g1'].reshape(1, 36), p['mlp_beta1'].reshape(1, 36),
          p['mlp_W2'], p['mlp_b2'].reshape(1, NUM_CLASSES)]
    return _head_call(acc, p['gat3_b'].reshape(1, WIDTH),
                      p['bn3_g'].reshape(1, WIDTH),
                      p['bn3_b'].reshape(1, WIDTH), batch3, mp, *cfg[3])


# speedup vs baseline: 1.0136x; 1.0097x over previous
import functools

import jax
import jax.numpy as jnp
from jax import lax
from jax.experimental import pallas as pl
from jax.experimental.pallas import tpu as pltpu
from jax.experimental.pallas import tpu_sc as plsc

N = 10000
F_IN = 128
WIDTH = 144
NUM_GRAPHS = 64
NUM_CLASSES = 112
TCOLS = 160
NROWS = 10240
E_RAW = 320000
E_TOT = E_RAW + N
SC_CORES = 2
SC_SUBCORES = 16
CHUNK = 48
EPW = 10368
NCH = EPW // CHUNK
EPAD = SC_CORES * SC_SUBCORES * EPW
ROWS_PT = NROWS // SC_SUBCORES
BLK = 400
GRID = N // BLK
EPS = 1e-5


def _edge_body(t_hbm, ad_hbm, src_hbm, dst_hbm, imap_hbm, acc_hbm,
               shared, tb0, tb1, tb2, ab0, ab1, ab2,
               is0a, is0b, is1a, is1b, is2a, is2b,
               id0a, id0b, id1a, id1b, id2a, id2b,
               iprime, imapv,
               st0, st1, st2, sa0, sa1, sa2, ss0, ss1, ss2, si0, si1, si2):
    cid = lax.axis_index("c")
    sid = lax.axis_index("s")
    row0 = sid * ROWS_PT
    tb = (tb0, tb1, tb2)
    ab = (ab0, ab1, ab2)
    isl = ((is0a, is0b), (is1a, is1b), (is2a, is2b))
    idl = ((id0a, id0b), (id1a, id1b), (id2a, id2b))
    st = (st0, st1, st2)
    sa = (sa0, sa1, sa2)
    ss = (ss0, ss1, ss2)
    si = (si0, si1, si2)

    def _zrow(r, carry):
        for j in range(TCOLS // 16):
            tb0[r, pl.ds(j * 16, 16)] = jnp.zeros((16,), jnp.float32)
            tb1[r, pl.ds(j * 16, 16)] = jnp.zeros((16,), jnp.float32)
            tb2[r, pl.ds(j * 16, 16)] = jnp.zeros((16,), jnp.float32)
        return carry
    lax.fori_loop(0, CHUNK, _zrow, 0)
    for j in range(CHUNK // 16):
        iprime[pl.ds(j * 16, 16)] = jnp.full((16,), N, jnp.int32)

    for k in range(ROWS_PT // CHUNK):
        pltpu.sync_copy(tb0, shared.at[pl.ds(row0 + k * CHUNK, CHUNK)])
    rem = ROWS_PT % CHUNK
    if rem:
        pltpu.sync_copy(tb0.at[pl.ds(0, rem)],
                        shared.at[pl.ds(row0 + (ROWS_PT // CHUNK) * CHUNK, rem)])
    plsc.subcore_barrier()

    pltpu.sync_copy(imap_hbm, imapv)
    imaps = [imapv[pl.ds(j * 16, 16)] for j in range(WIDTH // 16)]

    ebase = (cid * SC_SUBCORES + sid) * EPW

    def _compute(x):
        def _edge(e, c2):
            a = tb[x][e, pl.ds(WIDTH, 16)] + ab[x][e, :]
            s = jnp.exp(jnp.maximum(a, 0.0) + 0.2 * jnp.minimum(a, 0.0))
            tb[x][e, pl.ds(WIDTH, 16)] = s
            for j in range(WIDTH // 16):
                m = jnp.take_along_axis(s, imaps[j], axis=0,
                                        mode="promise_in_bounds")
                tb[x][e, pl.ds(j * 16, 16)] = m * tb[x][e, pl.ds(j * 16, 16)]
            return c2
        lax.fori_loop(0, CHUNK, _edge, 0)

    def _idx_copy(c, x, p, sem):
        b = ebase + c * CHUNK
        pltpu.async_copy(src_hbm.at[pl.ds(b, CHUNK)], isl[x][p], sem)
        pltpu.async_copy(dst_hbm.at[pl.ds(b, CHUNK)], idl[x][p], sem)

    def _idx_wait(x, p, sem):
        pltpu.make_async_copy(src_hbm.at[pl.ds(0, CHUNK)], isl[x][p], sem).wait()
        pltpu.make_async_copy(dst_hbm.at[pl.ds(0, CHUNK)], idl[x][p], sem).wait()

    def _gather(x, p):
        pltpu.async_copy(t_hbm.at[isl[x][p]], tb[x], st[x])
        pltpu.async_copy(ad_hbm.at[idl[x][p]], ab[x], sa[x])

    def _gather_wait(x):
        pltpu.make_async_copy(t_hbm.at[isl[x][0]], tb[x], st[x]).wait()
        pltpu.make_async_copy(ad_hbm.at[idl[x][0]], ab[x], sa[x]).wait()

    def _scatter(x, p, sem):
        pltpu.async_copy(tb[x], shared.at[idl[x][p]], sem, add=True)

    def _scatter_wait(x):
        pltpu.make_async_copy(tb[x], shared.at[iprime], ss[x]).wait()

    pltpu.sync_copy(src_hbm.at[pl.ds(ebase, CHUNK)], is0a)
    pltpu.sync_copy(dst_hbm.at[pl.ds(ebase, CHUNK)], id0a)
    pltpu.sync_copy(src_hbm.at[pl.ds(ebase + CHUNK, CHUNK)], is1a)
    pltpu.sync_copy(dst_hbm.at[pl.ds(ebase + CHUNK, CHUNK)], id1a)
    _idx_copy(2, 2, 0, si[2])
    _gather(0, 0)
    _gather(1, 0)

    def _visit(g, k, first=False):
        x = k % 3
        p = (k // 3) % 2
        prev = (x + 2) % 3
        p2 = ((k + 2) // 3) % 2
        _gather_wait(x)
        _idx_copy((g + 3) % NCH, x, 1 - p, si[x])
        _compute(x)
        _scatter(x, p, ss[x])
        if not first:
            _scatter_wait(prev)
        _idx_wait(prev, p2, si[prev])
        _gather(prev, p2)

    for k in range(6):
        _visit(k, k, first=(k == 0))

    def _six(i, carry):
        g0 = 6 + i * 6
        for k in range(6):
            _visit(g0 + k, k)
        return carry
    lax.fori_loop(0, NCH // 6 - 1, _six, 0)

    _gather_wait(0)
    _gather_wait(1)
    _scatter_wait(2)
    _idx_wait(2, 0, si[2])

    plsc.subcore_barrier()
    pltpu.sync_copy(shared.at[pl.ds(row0, ROWS_PT)],
                    acc_hbm.at[cid, pl.ds(row0, ROWS_PT), :])


@functools.lru_cache(maxsize=1)
def _build_edge_kernel():
    return functools.partial(
        pl.kernel,
        out_type=jax.ShapeDtypeStruct((SC_CORES, NROWS, TCOLS), jnp.float32),
        mesh=plsc.VectorSubcoreMesh(core_axis_name="c", subcore_axis_name="s",
                                    num_cores=SC_CORES,
                                    num_subcores=SC_SUBCORES),
        scratch_types=(
            [pltpu.VMEM_SHARED((NROWS, TCOLS), jnp.float32)]
            + [pltpu.VMEM((CHUNK, TCOLS), jnp.float32)] * 3
            + [pltpu.VMEM((CHUNK, 16), jnp.float32)] * 3
            + [pltpu.VMEM((CHUNK,), jnp.int32)] * 12
            + [pltpu.VMEM((CHUNK,), jnp.int32)]
            + [pltpu.VMEM((TCOLS,), jnp.int32)]
            + [pltpu.SemaphoreType.DMA] * 12
        ),
        compiler_params=pltpu.CompilerParams(use_tc_tiling_on_sc=False),
    )(_edge_body)


def _edge_call(t, ad16, src, dst, imap):
    return _build_edge_kernel()(t, ad16, src, dst, imap)


def _emit_tables(h, asf, adf, t_ref, ad_ref, heads, ch):
    col_head = lax.broadcasted_iota(jnp.int32, (WIDTH, heads), 0) // ch
    hid = lax.broadcasted_iota(jnp.int32, (WIDTH, heads), 1)
    m = (col_head == hid).astype(jnp.float32)
    as_ = jnp.dot(h * asf, m, preferred_element_type=jnp.float32,
                 precision=lax.Precision.HIGHEST)
    ad_ = jnp.dot(h * adf, m, preferred_element_type=jnp.float32,
                 precision=lax.Precision.HIGHEST)
    zp = jnp.zeros((h.shape[0], 16 - heads), jnp.float32)
    t_ref[...] = jnp.concatenate([h, as_, zp], axis=1)
    ad_ref[...] = jnp.concatenate([ad_, zp], axis=1)


def _pre0_body(x_ref, wemb_ref, bemb_ref, w_ref, asf_ref, adf_ref,
               t_ref, ad_ref, *, heads, ch):
    x0 = jnp.dot(x_ref[...], wemb_ref[...],
                 preferred_element_type=jnp.float32) + bemb_ref[...]
    h = jnp.dot(x0, w_ref[...], preferred_element_type=jnp.float32)
    _emit_tables(h, asf_ref[...], adf_ref[...], t_ref, ad_ref, heads, ch)


def _prei_body(z_ref, ps_ref, g_ref, b_ref, w_ref, asf_ref, adf_ref,
               t_ref, ad_ref, *, heads, ch):
    sums = jnp.sum(ps_ref[...][:, 0, :], axis=0, keepdims=True)
    mu = sums[:, :WIDTH] / N
    var = sums[:, WIDTH:] / N - mu * mu
    scale = g_ref[...] * lax.rsqrt(var + EPS)
    shift = b_ref[...] - mu * scale
    xn = z_ref[...] * scale + shift
    h = jnp.dot(xn, w_ref[...], preferred_element_type=jnp.float32)
    _emit_tables(h, asf_ref[...], adf_ref[...], t_ref, ad_ref, heads, ch)


def _post_body(acc_ref, bias_ref, z_ref, ps_ref, *, heads, ch):
    acc = acc_ref[0] + acc_ref[1]
    num = acc[:, :WIDTH]
    den = acc[:, WIDTH:WIDTH + heads]
    hid = lax.broadcasted_iota(jnp.int32, (heads, WIDTH), 0)
    col_head = lax.broadcasted_iota(jnp.int32, (heads, WIDTH), 1) // ch
    bexp = (hid == col_head).astype(jnp.float32)
    dex = jnp.dot(den, bexp, preferred_element_type=jnp.float32,
                 precision=lax.Precision.HIGHEST)
    z = jnp.maximum(num / (dex + 1e-16) + bias_ref[...], 0.0)
    z_ref[...] = z
    ps_ref[...] = jnp.concatenate(
        [jnp.sum(z, axis=0, keepdims=True),
         jnp.sum(z * z, axis=0, keepdims=True)], axis=1)[None]


def _bn_small(x, g, b):
    mu = jnp.mean(x, axis=0, keepdims=True)
    var = jnp.mean(x * x, axis=0, keepdims=True) - mu * mu
    return g * (x - mu) * lax.rsqrt(var + EPS) + b


def _head_body(z_ref, ps_ref, g_ref, b_ref, batch_ref,
               w0_ref, b0_ref, g0_ref, t0_ref,
               w1_ref, b1_ref, g1_ref, t1_ref,
               w2_ref, b2_ref, out_ref, pooled, cnt):
    i = pl.program_id(0)

    @pl.when(i == 0)
    def _init():
        pooled[...] = jnp.zeros_like(pooled)
        cnt[...] = jnp.zeros_like(cnt)

    sums = jnp.sum(ps_ref[...][:, 0, :], axis=0, keepdims=True)
    mu = sums[:, :WIDTH] / N
    var = sums[:, WIDTH:] / N - mu * mu
    scale = g_ref[...] * lax.rsqrt(var + EPS)
    shift = b_ref[...] - mu * scale
    zn = z_ref[...] * scale + shift

    bvec = batch_ref[0]
    gid = lax.broadcasted_iota(jnp.int32, (NUM_GRAPHS, BLK), 0)
    onehot_t = (gid == bvec).astype(jnp.float32)
    pooled[...] += jnp.dot(onehot_t, zn, preferred_element_type=jnp.float32,
                 precision=lax.Precision.HIGHEST)
    cnt[...] += jnp.dot(onehot_t, jnp.ones((BLK, 1), jnp.float32),
                        preferred_element_type=jnp.float32,
                 precision=lax.Precision.HIGHEST)

    @pl.when(i == GRID - 1)
    def _finish():
        pool = pooled[...] / jnp.maximum(cnt[...], 1.0)
        h0 = jnp.dot(pool, w0_ref[...],
                     preferred_element_type=jnp.float32) + b0_ref[...]
        h0 = jnp.maximum(_bn_small(h0, g0_ref[...], t0_ref[...]), 0.0)
        h1 = jnp.dot(h0, w1_ref[...],
                     preferred_element_type=jnp.float32) + b1_ref[...]
        h1 = jnp.maximum(_bn_small(h1, g1_ref[...], t1_ref[...]), 0.0)
        out_ref[...] = jnp.dot(h1, w2_ref[...],
                     preferred_element_type=jnp.float32) + b2_ref[...]


def _full(shape):
    return pl.BlockSpec(shape, lambda i: tuple(0 for _ in shape))


def _rows(cols):
    return pl.BlockSpec((BLK, cols), lambda i: (i, 0))


def _pre0_call(x, wemb, bemb, w, asf, adf, heads, ch):
    return pl.pallas_call(
        functools.partial(_pre0_body, heads=heads, ch=ch),
        grid=(GRID,),
        in_specs=[_rows(F_IN), _full((F_IN, WIDTH)), _full((1, WIDTH)),
                  _full((WIDTH, WIDTH)), _full((1, WIDTH)), _full((1, WIDTH))],
        out_specs=[_rows(TCOLS), _rows(16)],
        out_shape=[jax.ShapeDtypeStruct((N, TCOLS), jnp.float32),
                   jax.ShapeDtypeStruct((N, 16), jnp.float32)],
    )(x, wemb, bemb, w, asf, adf)


def _prei_call(z, ps, g, b, w, asf, adf, heads, ch):
    return pl.pallas_call(
        functools.partial(_prei_body, heads=heads, ch=ch),
        grid=(GRID,),
        in_specs=[_rows(WIDTH), _full((GRID, 1, 2 * WIDTH)), _full((1, WIDTH)),
                  _full((1, WIDTH)), _full((WIDTH, WIDTH)),
                  _full((1, WIDTH)), _full((1, WIDTH))],
        out_specs=[_rows(TCOLS), _rows(16)],
        out_shape=[jax.ShapeDtypeStruct((N, TCOLS), jnp.float32),
                   jax.ShapeDtypeStruct((N, 16), jnp.float32)],
    )(z, ps, g, b, w, asf, adf)


def _post_call(acc, bias, heads, ch):
    return pl.pallas_call(
        functools.partial(_post_body, heads=heads, ch=ch),
        grid=(GRID,),
        in_specs=[pl.BlockSpec((SC_CORES, BLK, TCOLS), lambda i: (0, i, 0)),
                  _full((1, WIDTH))],
        out_specs=[_rows(WIDTH),
                   pl.BlockSpec((1, 1, 2 * WIDTH), lambda i: (i, 0, 0))],
        out_shape=[jax.ShapeDtypeStruct((N, WIDTH), jnp.float32),
                   jax.ShapeDtypeStruct((GRID, 1, 2 * WIDTH), jnp.float32)],
    )(acc, bias)


def _head_call(z, ps, g, b, batch3, mp):
    return pl.pallas_call(
        _head_body,
        grid=(GRID,),
        in_specs=[_rows(WIDTH), _full((GRID, 1, 2 * WIDTH)), _full((1, WIDTH)),
                  _full((1, WIDTH)),
                  pl.BlockSpec((1, 1, BLK), lambda i: (i, 0, 0)),
                  _full((WIDTH, 72)), _full((1, 72)), _full((1, 72)),
                  _full((1, 72)),
                  _full((72, 36)), _full((1, 36)), _full((1, 36)),
                  _full((1, 36)),
                  _full((36, NUM_CLASSES)), _full((1, NUM_CLASSES))],
        out_specs=pl.BlockSpec((NUM_GRAPHS, NUM_CLASSES), lambda i: (0, 0)),
        out_shape=jax.ShapeDtypeStruct((NUM_GRAPHS, NUM_CLASSES), jnp.float32),
        scratch_shapes=[pltpu.VMEM((NUM_GRAPHS, WIDTH), jnp.float32),
                        pltpu.VMEM((NUM_GRAPHS, 1), jnp.float32)],
    )(z, ps, g, b, batch3, *mp)


def kernel(x, edge_index, batch, params):
    p = params
    loops = jnp.arange(N, dtype=jnp.int32)
    npad = EPAD - E_TOT
    src = jnp.concatenate([edge_index[0].astype(jnp.int32), loops,
                           jnp.zeros((npad,), jnp.int32)])
    dst = jnp.concatenate([edge_index[1].astype(jnp.int32), loops,
                           jnp.full((npad,), N, jnp.int32)])
    imap_multi = jnp.arange(TCOLS, dtype=jnp.int32) // 18
    imap_single = jnp.zeros((TCOLS,), jnp.int32)
    batch3 = batch.astype(jnp.int32).reshape(GRID, 1, BLK)

    cfg = [(8, 18), (8, 18), (8, 18), (1, WIDTH)]
    z = None
    ps = None
    for i, (heads, ch) in enumerate(cfg):
        asf = p[f'gat{i}_as'].reshape(1, WIDTH)
        adf = p[f'gat{i}_ad'].reshape(1, WIDTH)
        w = p[f'gat{i}_W']
        if i == 0:
            t, ad16 = _pre0_call(x, p['W_emb'], p['b_emb'].reshape(1, WIDTH),
                                 w, asf, adf, heads, ch)
        else:
            t, ad16 = _prei_call(z, ps, p[f'bn{i-1}_g'].reshape(1, WIDTH),
                                 p[f'bn{i-1}_b'].reshape(1, WIDTH),
                                 w, asf, adf, heads, ch)
        imap = imap_multi if heads > 1 else imap_single
        acc = _edge_call(t, ad16, src, dst, imap)
        z, ps = _post_call(acc, p[f'gat{i}_b'].reshape(1, WIDTH), heads, ch)

    mp = [p['mlp_W0'], p['mlp_b0'].reshape(1, 72),
          p['mlp_g0'].reshape(1, 72), p['mlp_beta0'].reshape(1, 72),
          p['mlp_W1'], p['mlp_b1'].reshape(1, 36),
          p['mlp_g1'].reshape(1, 36), p['mlp_beta1'].reshape(1, 36),
          p['mlp_W2'], p['mlp_b2'].reshape(1, NUM_CLASSES)]
    return _head_call(z, ps, p['bn3_g'].reshape(1, WIDTH),
                      p['bn3_b'].reshape(1, WIDTH), batch3, mp)


# speedup vs baseline: 1.0238x; 1.0101x over previous
import functools

import jax
import jax.numpy as jnp
from jax import lax
from jax.experimental import pallas as pl
from jax.experimental.pallas import tpu as pltpu
from jax.experimental.pallas import tpu_sc as plsc

N = 10000
F_IN = 128
WIDTH = 144
NUM_GRAPHS = 64
NUM_CLASSES = 112
TCOLS = 160
NROWS = 10240
E_RAW = 320000
E_TOT = E_RAW + N
SC_CORES = 2
SC_SUBCORES = 16
CHUNK = 48
EPW = 10368
NCH = EPW // CHUNK
EPAD = SC_CORES * SC_SUBCORES * EPW
ROWS_PT = NROWS // SC_SUBCORES
BLK = 400
GRID = N // BLK
EPS = 1e-5


def _edge_body(t_hbm, ad_hbm, src_hbm, dst_hbm, imap_hbm, acc_hbm,
               shared, tb0, tb1, tb2, ab0, ab1, ab2,
               is0a, is0b, is1a, is1b, is2a, is2b,
               id0a, id0b, id1a, id1b, id2a, id2b,
               iprime, imapv,
               st0, st1, st2, sa0, sa1, sa2, ss0, ss1, ss2, si0, si1, si2):
    cid = lax.axis_index("c")
    sid = lax.axis_index("s")
    row0 = sid * ROWS_PT
    tb = (tb0, tb1, tb2)
    ab = (ab0, ab1, ab2)
    isl = ((is0a, is0b), (is1a, is1b), (is2a, is2b))
    idl = ((id0a, id0b), (id1a, id1b), (id2a, id2b))
    st = (st0, st1, st2)
    sa = (sa0, sa1, sa2)
    ss = (ss0, ss1, ss2)
    si = (si0, si1, si2)

    def _zrow(r, carry):
        for j in range(TCOLS // 16):
            tb2[r, pl.ds(j * 16, 16)] = jnp.zeros((16,), jnp.float32)
        return carry
    lax.fori_loop(0, CHUNK, _zrow, 0)
    for j in range(CHUNK // 16):
        iprime[pl.ds(j * 16, 16)] = jnp.full((16,), N, jnp.int32)

    nz = ROWS_PT // CHUNK
    rem = ROWS_PT % CHUNK
    for k in range(nz):
        pltpu.async_copy(tb2, shared.at[pl.ds(row0 + k * CHUNK, CHUNK)], ss[2])
    if rem:
        pltpu.async_copy(tb2.at[pl.ds(0, rem)],
                         shared.at[pl.ds(row0 + nz * CHUNK, rem)], ss[2])

    pltpu.sync_copy(imap_hbm, imapv)
    imaps = [imapv[pl.ds(j * 16, 16)] for j in range(WIDTH // 16)]

    ebase = (cid * SC_SUBCORES + sid) * EPW

    def _compute(x):
        def _edge(e, c2):
            a = tb[x][e, pl.ds(WIDTH, 16)] + ab[x][e, :]
            s = jnp.exp(jnp.maximum(a, 0.0) + 0.2 * jnp.minimum(a, 0.0))
            tb[x][e, pl.ds(WIDTH, 16)] = s
            for j in range(WIDTH // 16):
                m = jnp.take_along_axis(s, imaps[j], axis=0,
                                        mode="promise_in_bounds")
                tb[x][e, pl.ds(j * 16, 16)] = m * tb[x][e, pl.ds(j * 16, 16)]
            return c2
        lax.fori_loop(0, CHUNK, _edge, 0)

    def _idx_copy(c, x, p, sem):
        b = ebase + c * CHUNK
        pltpu.async_copy(src_hbm.at[pl.ds(b, CHUNK)], isl[x][p], sem)
        pltpu.async_copy(dst_hbm.at[pl.ds(b, CHUNK)], idl[x][p], sem)

    def _idx_wait(x, p, sem):
        pltpu.make_async_copy(src_hbm.at[pl.ds(0, CHUNK)], isl[x][p], sem).wait()
        pltpu.make_async_copy(dst_hbm.at[pl.ds(0, CHUNK)], idl[x][p], sem).wait()

    def _gather(x, p):
        pltpu.async_copy(t_hbm.at[isl[x][p]], tb[x], st[x])
        pltpu.async_copy(ad_hbm.at[idl[x][p]], ab[x], sa[x])

    def _gather_wait(x):
        pltpu.make_async_copy(t_hbm.at[isl[x][0]], tb[x], st[x]).wait()
        pltpu.make_async_copy(ad_hbm.at[idl[x][0]], ab[x], sa[x]).wait()

    def _scatter(x, p, sem):
        pltpu.async_copy(tb[x], shared.at[idl[x][p]], sem, add=True)

    def _scatter_wait(x):
        pltpu.make_async_copy(tb[x], shared.at[iprime], ss[x]).wait()

    pltpu.sync_copy(src_hbm.at[pl.ds(ebase, CHUNK)], is0a)
    pltpu.sync_copy(dst_hbm.at[pl.ds(ebase, CHUNK)], id0a)
    pltpu.sync_copy(src_hbm.at[pl.ds(ebase + CHUNK, CHUNK)], is1a)
    pltpu.sync_copy(dst_hbm.at[pl.ds(ebase + CHUNK, CHUNK)], id1a)
    _idx_copy(2, 2, 0, si[2])
    _gather(0, 0)
    _gather(1, 0)

    for k in range(nz):
        pltpu.make_async_copy(
            tb2, shared.at[pl.ds(row0 + k * CHUNK, CHUNK)], ss[2]).wait()
    if rem:
        pltpu.make_async_copy(
            tb2.at[pl.ds(0, rem)],
            shared.at[pl.ds(row0 + nz * CHUNK, rem)], ss[2]).wait()
    plsc.subcore_barrier()

    def _visit(g, k, first=False):
        x = k % 3
        p = (k // 3) % 2
        prev = (x + 2) % 3
        p2 = ((k + 2) // 3) % 2
        _gather_wait(x)
        _idx_copy((g + 3) % NCH, x, 1 - p, si[x])
        _compute(x)
        _scatter(x, p, ss[x])
        if not first:
            _scatter_wait(prev)
        _idx_wait(prev, p2, si[prev])
        _gather(prev, p2)

    for k in range(6):
        _visit(k, k, first=(k == 0))

    def _six(i, carry):
        g0 = 6 + i * 6
        for k in range(6):
            _visit(g0 + k, k)
        return carry
    lax.fori_loop(0, NCH // 6 - 1, _six, 0)

    _gather_wait(0)
    _gather_wait(1)
    _scatter_wait(2)
    _idx_wait(2, 0, si[2])

    plsc.subcore_barrier()
    pltpu.sync_copy(shared.at[pl.ds(row0, ROWS_PT)],
                    acc_hbm.at[cid, pl.ds(row0, ROWS_PT), :])


@functools.lru_cache(maxsize=1)
def _build_edge_kernel():
    return functools.partial(
        pl.kernel,
        out_type=jax.ShapeDtypeStruct((SC_CORES, NROWS, TCOLS), jnp.float32),
        mesh=plsc.VectorSubcoreMesh(core_axis_name="c", subcore_axis_name="s",
                                    num_cores=SC_CORES,
                                    num_subcores=SC_SUBCORES),
        scratch_types=(
            [pltpu.VMEM_SHARED((NROWS, TCOLS), jnp.float32)]
            + [pltpu.VMEM((CHUNK, TCOLS), jnp.float32)] * 3
            + [pltpu.VMEM((CHUNK, 16), jnp.float32)] * 3
            + [pltpu.VMEM((CHUNK,), jnp.int32)] * 12
            + [pltpu.VMEM((CHUNK,), jnp.int32)]
            + [pltpu.VMEM((TCOLS,), jnp.int32)]
            + [pltpu.SemaphoreType.DMA] * 12
        ),
        compiler_params=pltpu.CompilerParams(use_tc_tiling_on_sc=False),
    )(_edge_body)


def _edge_call(t, ad16, src, dst, imap):
    return _build_edge_kernel()(t, ad16, src, dst, imap)


def _emit_tables(h, asf, adf, t_ref, ad_ref, heads, ch):
    col_head = lax.broadcasted_iota(jnp.int32, (WIDTH, heads), 0) // ch
    hid = lax.broadcasted_iota(jnp.int32, (WIDTH, heads), 1)
    m = (col_head == hid).astype(jnp.float32)
    as_ = jnp.dot(h * asf, m, preferred_element_type=jnp.float32,
                 precision=lax.Precision.HIGHEST)
    ad_ = jnp.dot(h * adf, m, preferred_element_type=jnp.float32,
                 precision=lax.Precision.HIGHEST)
    zp = jnp.zeros((h.shape[0], 16 - heads), jnp.float32)
    t_ref[...] = jnp.concatenate([h, as_, zp], axis=1)
    ad_ref[...] = jnp.concatenate([ad_, zp], axis=1)


def _pre0_body(x_ref, wemb_ref, bemb_ref, w_ref, asf_ref, adf_ref,
               t_ref, ad_ref, *, heads, ch):
    x0 = jnp.dot(x_ref[...], wemb_ref[...],
                 preferred_element_type=jnp.float32) + bemb_ref[...]
    h = jnp.dot(x0, w_ref[...], preferred_element_type=jnp.float32)
    _emit_tables(h, asf_ref[...], adf_ref[...], t_ref, ad_ref, heads, ch)


def _prei_body(z_ref, ps_ref, g_ref, b_ref, w_ref, asf_ref, adf_ref,
               t_ref, ad_ref, *, heads, ch):
    sums = jnp.sum(ps_ref[...][:, 0, :], axis=0, keepdims=True)
    mu = sums[:, :WIDTH] / N
    var = sums[:, WIDTH:] / N - mu * mu
    scale = g_ref[...] * lax.rsqrt(var + EPS)
    shift = b_ref[...] - mu * scale
    xn = z_ref[...] * scale + shift
    h = jnp.dot(xn, w_ref[...], preferred_element_type=jnp.float32)
    _emit_tables(h, asf_ref[...], adf_ref[...], t_ref, ad_ref, heads, ch)


def _post_body(acc_ref, bias_ref, z_ref, ps_ref, *, heads, ch):
    acc = acc_ref[0] + acc_ref[1]
    num = acc[:, :WIDTH]
    den = acc[:, WIDTH:WIDTH + heads]
    hid = lax.broadcasted_iota(jnp.int32, (heads, WIDTH), 0)
    col_head = lax.broadcasted_iota(jnp.int32, (heads, WIDTH), 1) // ch
    bexp = (hid == col_head).astype(jnp.float32)
    dex = jnp.dot(den, bexp, preferred_element_type=jnp.float32,
                 precision=lax.Precision.HIGHEST)
    z = jnp.maximum(num / (dex + 1e-16) + bias_ref[...], 0.0)
    z_ref[...] = z
    ps_ref[...] = jnp.concatenate(
        [jnp.sum(z, axis=0, keepdims=True),
         jnp.sum(z * z, axis=0, keepdims=True)], axis=1)[None]


def _bn_small(x, g, b):
    mu = jnp.mean(x, axis=0, keepdims=True)
    var = jnp.mean(x * x, axis=0, keepdims=True) - mu * mu
    return g * (x - mu) * lax.rsqrt(var + EPS) + b


def _head_body(z_ref, ps_ref, g_ref, b_ref, batch_ref,
               w0_ref, b0_ref, g0_ref, t0_ref,
               w1_ref, b1_ref, g1_ref, t1_ref,
               w2_ref, b2_ref, out_ref, pooled, cnt):
    i = pl.program_id(0)

    @pl.when(i == 0)
    def _init():
        pooled[...] = jnp.zeros_like(pooled)
        cnt[...] = jnp.zeros_like(cnt)

    sums = jnp.sum(ps_ref[...][:, 0, :], axis=0, keepdims=True)
    mu = sums[:, :WIDTH] / N
    var = sums[:, WIDTH:] / N - mu * mu
    scale = g_ref[...] * lax.rsqrt(var + EPS)
    shift = b_ref[...] - mu * scale
    zn = z_ref[...] * scale + shift

    bvec = batch_ref[0]
    gid = lax.broadcasted_iota(jnp.int32, (NUM_GRAPHS, BLK), 0)
    onehot_t = (gid == bvec).astype(jnp.float32)
    pooled[...] += jnp.dot(onehot_t, zn, preferred_element_type=jnp.float32,
                 precision=lax.Precision.HIGHEST)
    cnt[...] += jnp.dot(onehot_t, jnp.ones((BLK, 1), jnp.float32),
                        preferred_element_type=jnp.float32,
                 precision=lax.Precision.HIGHEST)

    @pl.when(i == GRID - 1)
    def _finish():
        pool = pooled[...] / jnp.maximum(cnt[...], 1.0)
        h0 = jnp.dot(pool, w0_ref[...],
                     preferred_element_type=jnp.float32) + b0_ref[...]
        h0 = jnp.maximum(_bn_small(h0, g0_ref[...], t0_ref[...]), 0.0)
        h1 = jnp.dot(h0, w1_ref[...],
                     preferred_element_type=jnp.float32) + b1_ref[...]
        h1 = jnp.maximum(_bn_small(h1, g1_ref[...], t1_ref[...]), 0.0)
        out_ref[...] = jnp.dot(h1, w2_ref[...],
                     preferred_element_type=jnp.float32) + b2_ref[...]


def _full(shape):
    return pl.BlockSpec(shape, lambda i: tuple(0 for _ in shape))


def _rows(cols):
    return pl.BlockSpec((BLK, cols), lambda i: (i, 0))


def _pre0_call(x, wemb, bemb, w, asf, adf, heads, ch):
    return pl.pallas_call(
        functools.partial(_pre0_body, heads=heads, ch=ch),
        grid=(GRID,),
        in_specs=[_rows(F_IN), _full((F_IN, WIDTH)), _full((1, WIDTH)),
                  _full((WIDTH, WIDTH)), _full((1, WIDTH)), _full((1, WIDTH))],
        out_specs=[_rows(TCOLS), _rows(16)],
        out_shape=[jax.ShapeDtypeStruct((N, TCOLS), jnp.float32),
                   jax.ShapeDtypeStruct((N, 16), jnp.float32)],
    )(x, wemb, bemb, w, asf, adf)


def _prei_call(z, ps, g, b, w, asf, adf, heads, ch):
    return pl.pallas_call(
        functools.partial(_prei_body, heads=heads, ch=ch),
        grid=(GRID,),
        in_specs=[_rows(WIDTH), _full((GRID, 1, 2 * WIDTH)), _full((1, WIDTH)),
                  _full((1, WIDTH)), _full((WIDTH, WIDTH)),
                  _full((1, WIDTH)), _full((1, WIDTH))],
        out_specs=[_rows(TCOLS), _rows(16)],
        out_shape=[jax.ShapeDtypeStruct((N, TCOLS), jnp.float32),
                   jax.ShapeDtypeStruct((N, 16), jnp.float32)],
    )(z, ps, g, b, w, asf, adf)


def _post_call(acc, bias, heads, ch):
    return pl.pallas_call(
        functools.partial(_post_body, heads=heads, ch=ch),
        grid=(GRID,),
        in_specs=[pl.BlockSpec((SC_CORES, BLK, TCOLS), lambda i: (0, i, 0)),
                  _full((1, WIDTH))],
        out_specs=[_rows(WIDTH),
                   pl.BlockSpec((1, 1, 2 * WIDTH), lambda i: (i, 0, 0))],
        out_shape=[jax.ShapeDtypeStruct((N, WIDTH), jnp.float32),
                   jax.ShapeDtypeStruct((GRID, 1, 2 * WIDTH), jnp.float32)],
    )(acc, bias)


def _head_call(z, ps, g, b, batch3, mp):
    return pl.pallas_call(
        _head_body,
        grid=(GRID,),
        in_specs=[_rows(WIDTH), _full((GRID, 1, 2 * WIDTH)), _full((1, WIDTH)),
                  _full((1, WIDTH)),
                  pl.BlockSpec((1, 1, BLK), lambda i: (i, 0, 0)),
                  _full((WIDTH, 72)), _full((1, 72)), _full((1, 72)),
                  _full((1, 72)),
                  _full((72, 36)), _full((1, 36)), _full((1, 36)),
                  _full((1, 36)),
                  _full((36, NUM_CLASSES)), _full((1, NUM_CLASSES))],
        out_specs=pl.BlockSpec((NUM_GRAPHS, NUM_CLASSES), lambda i: (0, 0)),
        out_shape=jax.ShapeDtypeStruct((NUM_GRAPHS, NUM_CLASSES), jnp.float32),
        scratch_shapes=[pltpu.VMEM((NUM_GRAPHS, WIDTH), jnp.float32),
                        pltpu.VMEM((NUM_GRAPHS, 1), jnp.float32)],
    )(z, ps, g, b, batch3, *mp)


def kernel(x, edge_index, batch, params):
    p = params
    loops = jnp.arange(N, dtype=jnp.int32)
    npad = EPAD - E_TOT
    src = jnp.concatenate([edge_index[0].astype(jnp.int32), loops,
                           jnp.zeros((npad,), jnp.int32)])
    dst = jnp.concatenate([edge_index[1].astype(jnp.int32), loops,
                           jnp.full((npad,), N, jnp.int32)])
    imap_multi = jnp.arange(TCOLS, dtype=jnp.int32) // 18
    imap_single = jnp.zeros((TCOLS,), jnp.int32)
    batch3 = batch.astype(jnp.int32).reshape(GRID, 1, BLK)

    cfg = [(8, 18), (8, 18), (8, 18), (1, WIDTH)]
    z = None
    ps = None
    for i, (heads, ch) in enumerate(cfg):
        asf = p[f'gat{i}_as'].reshape(1, WIDTH)
        adf = p[f'gat{i}_ad'].reshape(1, WIDTH)
        w = p[f'gat{i}_W']
        if i == 0:
            t, ad16 = _pre0_call(x, p['W_emb'], p['b_emb'].reshape(1, WIDTH),
                                 w, asf, adf, heads, ch)
        else:
            t, ad16 = _prei_call(z, ps, p[f'bn{i-1}_g'].reshape(1, WIDTH),
                                 p[f'bn{i-1}_b'].reshape(1, WIDTH),
                                 w, asf, adf, heads, ch)
        imap = imap_multi if heads > 1 else imap_single
        acc = _edge_call(t, ad16, src, dst, imap)
        z, ps = _post_call(acc, p[f'gat{i}_b'].reshape(1, WIDTH), heads, ch)

    mp = [p['mlp_W0'], p['mlp_b0'].reshape(1, 72),
          p['mlp_g0'].reshape(1, 72), p['mlp_beta0'].reshape(1, 72),
          p['mlp_W1'], p['mlp_b1'].reshape(1, 36),
          p['mlp_g1'].reshape(1, 36), p['mlp_beta1'].reshape(1, 36),
          p['mlp_W2'], p['mlp_b2'].reshape(1, NUM_CLASSES)]
    return _head_call(z, ps, p['bn3_g'].reshape(1, WIDTH),
                      p['bn3_b'].reshape(1, WIDTH), batch3, mp)
